# Initial kernel scaffold; baseline (speedup 1.0000x reference)
import jax
import jax.numpy as jnp
from jax.experimental import pallas as pl


def kernel(x, edge_index, W1_l, b1, W1_r, W2_l, b2, W2_r):
    raise NotImplementedError("write your pallas kernel here")


# speedup vs baseline: 13.5291x; 13.5291x over previous
import jax
import jax.numpy as jnp
from jax import lax
from jax.experimental import pallas as pl
from jax.experimental.pallas import tpu as pltpu
from jax.experimental.pallas import tpu_sc as plsc

N = 10000
D = 128
E = 320000
NC = 2
NS = 16
NW = NC * NS
C = 128
NCHUNK = 80
EPT = C * NCHUNK
EPAD = NW * EPT
NPAD = 10240
RPT = NPAD // NS


def _make_edge_pass(with_deg):
  mesh = plsc.VectorSubcoreMesh(core_axis_name="c", subcore_axis_name="s")
  out_type = [jax.ShapeDtypeStruct((NC * NPAD, D), jnp.float32)]
  scratch = [
      pltpu.VMEM((4, C), jnp.int32),
      pltpu.VMEM((4, C), jnp.int32),
      pltpu.VMEM((C, D), jnp.float32),
      pltpu.VMEM((C, D), jnp.float32),
      pltpu.VMEM_SHARED((NPAD, D), jnp.float32),
      pltpu.SemaphoreType.DMA,
      pltpu.SemaphoreType.DMA,
      pltpu.SemaphoreType.DMA,
      pltpu.SemaphoreType.DMA,
      pltpu.SemaphoreType.DMA,
      pltpu.SemaphoreType.DMA,
  ]
  if with_deg:
    out_type.append(jax.ShapeDtypeStruct((NC * NPAD,), jnp.float32))
    scratch += [
        pltpu.VMEM((C,), jnp.float32),
        pltpu.VMEM((RPT,), jnp.float32),
        pltpu.VMEM_SHARED((NPAD,), jnp.float32),
    ]

  def body(table, srcs, dsts, *refs):
    if with_deg:
      (out_p, out_deg, si, di, buf0, buf1, acc, sem0, sem1,
       is0, is1, is2, is3, ones_v, zeros_v, deg_acc) = refs
    else:
      (out_p, si, di, buf0, buf1, acc, sem0, sem1,
       is0, is1, is2, is3) = refs
    isem = (is0, is1, is2, is3)
    gsem = (sem0, sem1)
    gbuf = (buf0, buf1)
    cid = lax.axis_index("c")
    sid = lax.axis_index("s")
    wid = sid * NC + cid
    base = wid * NCHUNK
    zvec = jnp.zeros((16,), jnp.float32)

    def zero_buf0(t, carry):
      buf0[t // 8, pl.ds((t % 8) * 16, 16)] = zvec
      return carry
    lax.fori_loop(0, C * 8, zero_buf0, 0)
    for b in range(RPT // C):
      pltpu.sync_copy(buf0, acc.at[pl.ds(sid * RPT + b * C, C)])

    if with_deg:
      def init_ones(t, carry):
        ones_v[pl.ds(t * 16, 16)] = zvec + 1.0
        return carry
      lax.fori_loop(0, C // 16, init_ones, 0)

      def zero_zeros(t, carry):
        zeros_v[pl.ds(t * 16, 16)] = zvec
        return carry
      lax.fori_loop(0, RPT // 16, zero_zeros, 0)
      pltpu.sync_copy(zeros_v, deg_acc.at[pl.ds(sid * RPT, RPT)])

    def start_idx(c, slot):
      pltpu.async_copy(srcs.at[base + c], si.at[slot], isem[slot])
      pltpu.async_copy(dsts.at[base + c], di.at[slot], isem[slot])

    def wait_idx(slot):
      pltpu.make_async_copy(srcs.at[base], si.at[slot], isem[slot]).wait()
      pltpu.make_async_copy(dsts.at[base], di.at[slot], isem[slot]).wait()

    def start_gather(c, slot, b):
      pltpu.async_copy(table.at[si.at[slot]], gbuf[b], gsem[b])

    def wait_gather(b):
      pltpu.make_async_copy(table.at[si.at[0]], gbuf[b], gsem[b]).wait()

    def scatter(slot, b):
      pltpu.sync_copy(gbuf[b], acc.at[di.at[slot]], add=True)
      if with_deg:
        pltpu.sync_copy(ones_v, deg_acc.at[di.at[slot]], add=True)

    for c in range(4):
      start_idx(c, c)
    plsc.subcore_barrier()
    wait_idx(0)
    start_gather(0, 0, 0)
    wait_idx(1)
    start_gather(1, 1, 1)

    def quad(q, prefetch):
      c0 = 4 * q
      for k in range(4):
        b = k & 1
        wait_gather(b)
        scatter(k, b)
        if prefetch:
          start_idx(c0 + 4 + k, k)
        if k < 2:
          wait_idx(k + 2)
          start_gather(c0 + 2 + k, k + 2, b)
        elif prefetch:
          wait_idx(k - 2)
          start_gather(c0 + 2 + k, k - 2, b)

    lax.fori_loop(0, NCHUNK // 4 - 1, lambda q, c: (quad(q, True), c)[1], 0)
    quad(NCHUNK // 4 - 1, False)

    plsc.subcore_barrier()
    row0 = cid * NPAD + sid * RPT
    pltpu.sync_copy(acc.at[pl.ds(sid * RPT, RPT)],
                    out_p.at[pl.ds(row0, RPT)])
    if with_deg:
      pltpu.sync_copy(deg_acc.at[pl.ds(sid * RPT, RPT)],
                      out_deg.at[pl.ds(row0, RPT)])

  return pl.kernel(body, out_type=tuple(out_type), mesh=mesh,
                   scratch_types=tuple(scratch))


_edge_pass_deg = _make_edge_pass(True)
_edge_pass = _make_edge_pass(False)

_R = 1280


def _make_combine(apply_relu):
  def body(p_ref, deg_ref, x_ref, wl_ref, wr_ref, b_ref, o_ref):
    deg = deg_ref[:, 0:1] + deg_ref[:, 1:2]
    invd = 1.0 / jnp.maximum(deg, 1.0)
    mean = (p_ref[0] + p_ref[1]) * invd
    acc = jnp.dot(mean, wl_ref[...], preferred_element_type=jnp.float32)
    acc = acc + jnp.dot(x_ref[...], wr_ref[...],
                        preferred_element_type=jnp.float32)
    acc = acc + b_ref[...]
    if apply_relu:
      acc = jnp.maximum(acc, 0.0)
    o_ref[...] = acc

  return pl.pallas_call(
      body,
      grid=(NPAD // _R,),
      in_specs=[
          pl.BlockSpec((2, _R, D), lambda i: (0, i, 0)),
          pl.BlockSpec((_R, 2), lambda i: (i, 0)),
          pl.BlockSpec((_R, D), lambda i: (i, 0)),
          pl.BlockSpec((D, D), lambda i: (0, 0)),
          pl.BlockSpec((D, D), lambda i: (0, 0)),
          pl.BlockSpec((1, D), lambda i: (0, 0)),
      ],
      out_specs=pl.BlockSpec((_R, D), lambda i: (i, 0)),
      out_shape=jax.ShapeDtypeStruct((NPAD, D), jnp.float32),
  )


_combine_relu = _make_combine(True)
_combine_lin = _make_combine(False)


def kernel(x, edge_index, W1_l, b1, W1_r, W2_l, b2, W2_r):
  src = edge_index[0]
  dst = edge_index[1]
  pad = EPAD - E
  ar = jnp.arange(pad, dtype=jnp.int32)
  psrc = ar % N
  pdst = N + (ar % (NPAD - N))
  srcs = jnp.concatenate([src, psrc]).reshape(NW * NCHUNK, C)
  dsts = jnp.concatenate([dst, pdst]).reshape(NW * NCHUNK, C)
  xp = jnp.concatenate([x, jnp.zeros((NPAD - N, D), jnp.float32)], axis=0)

  p1, deg = _edge_pass_deg(x, srcs, dsts)
  degt = deg.reshape(NC, NPAD).T
  h = _combine_relu(p1.reshape(NC, NPAD, D), degt, xp,
                    W1_l, W1_r, b1.reshape(1, D))
  (p2,) = _edge_pass(h, srcs, dsts)
  z = _combine_lin(p2.reshape(NC, NPAD, D), degt, h,
                   W2_l, W2_r, b2.reshape(1, D))
  return z[:N]


# speedup vs baseline: 14.6899x; 1.0858x over previous
import jax
import jax.numpy as jnp
from jax import lax
from jax.experimental import pallas as pl
from jax.experimental.pallas import tpu as pltpu
from jax.experimental.pallas import tpu_sc as plsc

N = 10000
D = 128
E = 320000
NC = 2
NS = 16
NW = NC * NS
C = 80
NCHUNK = 128
EPT = C * NCHUNK
EPAD = NW * EPT
NPAD = 10240
RPT = NPAD // NS
NB = 4
ND = 8


def _make_edge_pass(with_deg):
  mesh = plsc.VectorSubcoreMesh(core_axis_name="c", subcore_axis_name="s")
  out_type = [jax.ShapeDtypeStruct((NC * NPAD, D), jnp.float32)]
  scratch = [
      pltpu.VMEM((NB, C), jnp.int32),
      pltpu.VMEM((ND, C), jnp.int32),
      pltpu.VMEM((NB, C, D), jnp.float32),
      pltpu.VMEM_SHARED((NPAD, D), jnp.float32),
  ]
  scratch += [pltpu.SemaphoreType.DMA] * NB
  scratch += [pltpu.SemaphoreType.DMA] * NB
  scratch += [pltpu.SemaphoreType.DMA] * NB
  scratch += [pltpu.SemaphoreType.DMA] * ND
  if with_deg:
    out_type.append(jax.ShapeDtypeStruct((NC * NPAD,), jnp.float32))
    scratch += [
        pltpu.VMEM((C,), jnp.float32),
        pltpu.VMEM((RPT,), jnp.float32),
        pltpu.VMEM_SHARED((NPAD,), jnp.float32),
    ]

  def body(table, srcs, dsts, *refs):
    out_p = refs[0]
    refs = refs[1:]
    if with_deg:
      out_deg = refs[0]
      refs = refs[1:]
    si, di, bufs, acc = refs[0:4]
    gsem = refs[4:4 + NB]
    ssem = refs[4 + NB:4 + 2 * NB]
    isem_s = refs[4 + 2 * NB:4 + 3 * NB]
    isem_d = refs[4 + 3 * NB:4 + 3 * NB + ND]
    if with_deg:
      ones_v, zeros_v, deg_acc = refs[4 + 3 * NB + ND:]
    cid = lax.axis_index("c")
    sid = lax.axis_index("s")
    wid = sid * NC + cid
    base = wid * NCHUNK
    zvec = jnp.zeros((16,), jnp.float32)

    def zero_buf0(t, carry):
      bufs[0, t // 8, pl.ds((t % 8) * 16, 16)] = zvec
      return carry
    lax.fori_loop(0, C * 8, zero_buf0, 0)
    for b in range(RPT // C):
      pltpu.sync_copy(bufs.at[0], acc.at[pl.ds(sid * RPT + b * C, C)])

    if with_deg:
      def init_ones(t, carry):
        ones_v[pl.ds(t * 16, 16)] = zvec + 1.0
        return carry
      lax.fori_loop(0, C // 16, init_ones, 0)

      def zero_zeros(t, carry):
        zeros_v[pl.ds(t * 16, 16)] = zvec
        return carry
      lax.fori_loop(0, RPT // 16, zero_zeros, 0)
      pltpu.sync_copy(zeros_v, deg_acc.at[pl.ds(sid * RPT, RPT)])

    def req_si(c, slot):
      pltpu.async_copy(srcs.at[base + c], si.at[slot], isem_s[slot])

    def wait_si(slot):
      pltpu.make_async_copy(srcs.at[base], si.at[slot], isem_s[slot]).wait()

    def req_di(c, slot):
      pltpu.async_copy(dsts.at[base + c], di.at[slot], isem_d[slot])

    def wait_di(slot):
      pltpu.make_async_copy(dsts.at[base], di.at[slot], isem_d[slot]).wait()

    def start_gather(c, slot):
      pltpu.async_copy(table.at[si.at[slot]], bufs.at[slot], gsem[slot])

    def wait_gather(slot):
      pltpu.make_async_copy(table.at[si.at[slot]], bufs.at[slot],
                            gsem[slot]).wait()

    def start_scatter(sb, dslot):
      pltpu.async_copy(bufs.at[sb], acc.at[di.at[dslot]], ssem[sb], add=True)
      if with_deg:
        pltpu.async_copy(ones_v, deg_acc.at[di.at[dslot]], ssem[sb], add=True)

    def wait_scatter(sb, dslot):
      pltpu.make_async_copy(bufs.at[sb], acc.at[di.at[dslot]],
                            ssem[sb]).wait()
      if with_deg:
        pltpu.make_async_copy(ones_v, deg_acc.at[di.at[dslot]],
                              ssem[sb]).wait()

    def full_step(c, k, first, tail_k=None):
      sb = k & 3
      wait_di(k & 7)
      wait_gather(sb)
      start_scatter(sb, k & 7)
      if tail_k is None or k <= 4:
        wait_si((k + 3) & 3)
        if not (first and k == 0):
          wait_scatter((k + 3) & 3, (k - 1) & 7)
        start_gather(c + 3, (k + 3) & 3)
      if tail_k is None or k <= 3:
        req_si(c + 4, k & 3)
      if (tail_k is None and not (first and k == 0)) or (
          tail_k is not None and k == 0):
        req_di(c + 7, (k - 1) & 7)

    for c in range(NB):
      req_si(c, c)
    for c in range(ND):
      req_di(c, c)
    plsc.subcore_barrier()
    for c in range(3):
      wait_si(c)
      start_gather(c, c)

    for k in range(8):
      full_step(k, k, first=True)

    def octet(o, carry):
      c0 = 8 * o
      for k in range(8):
        full_step(c0 + k, k, first=False)
      return carry
    lax.fori_loop(1, NCHUNK // 8 - 1, octet, 0)

    c0 = NCHUNK - 8
    for k in range(8):
      full_step(c0 + k, k, first=False, tail_k=k)

    for k in range(4, 8):
      wait_scatter(k & 3, k)

    plsc.subcore_barrier()
    row0 = cid * NPAD + sid * RPT
    pltpu.sync_copy(acc.at[pl.ds(sid * RPT, RPT)],
                    out_p.at[pl.ds(row0, RPT)])
    if with_deg:
      pltpu.sync_copy(deg_acc.at[pl.ds(sid * RPT, RPT)],
                      out_deg.at[pl.ds(row0, RPT)])

  return pl.kernel(body, out_type=tuple(out_type), mesh=mesh,
                   scratch_types=tuple(scratch))


_edge_pass_deg = _make_edge_pass(True)
_edge_pass = _make_edge_pass(False)

_R = 1280


def _make_combine(apply_relu):
  def body(p_ref, deg_ref, x_ref, wl_ref, wr_ref, b_ref, o_ref):
    deg = deg_ref[:, 0:1] + deg_ref[:, 1:2]
    invd = 1.0 / jnp.maximum(deg, 1.0)
    mean = (p_ref[0] + p_ref[1]) * invd
    acc = jnp.dot(mean, wl_ref[...], preferred_element_type=jnp.float32)
    acc = acc + jnp.dot(x_ref[...], wr_ref[...],
                        preferred_element_type=jnp.float32)
    acc = acc + b_ref[...]
    if apply_relu:
      acc = jnp.maximum(acc, 0.0)
    o_ref[...] = acc

  return pl.pallas_call(
      body,
      grid=(NPAD // _R,),
      in_specs=[
          pl.BlockSpec((2, _R, D), lambda i: (0, i, 0)),
          pl.BlockSpec((_R, 2), lambda i: (i, 0)),
          pl.BlockSpec((_R, D), lambda i: (i, 0)),
          pl.BlockSpec((D, D), lambda i: (0, 0)),
          pl.BlockSpec((D, D), lambda i: (0, 0)),
          pl.BlockSpec((1, D), lambda i: (0, 0)),
      ],
      out_specs=pl.BlockSpec((_R, D), lambda i: (i, 0)),
      out_shape=jax.ShapeDtypeStruct((NPAD, D), jnp.float32),
  )


_combine_relu = _make_combine(True)
_combine_lin = _make_combine(False)


def kernel(x, edge_index, W1_l, b1, W1_r, W2_l, b2, W2_r):
  src = edge_index[0]
  dst = edge_index[1]
  pad = EPAD - E
  ar = jnp.arange(pad, dtype=jnp.int32)
  psrc = ar % N
  pdst = N + (ar % (NPAD - N))
  srcs = jnp.concatenate([src, psrc]).reshape(NW * NCHUNK, C)
  dsts = jnp.concatenate([dst, pdst]).reshape(NW * NCHUNK, C)
  xp = jnp.concatenate([x, jnp.zeros((NPAD - N, D), jnp.float32)], axis=0)

  p1, deg = _edge_pass_deg(x, srcs, dsts)
  degt = deg.reshape(NC, NPAD).T
  h = _combine_relu(p1.reshape(NC, NPAD, D), degt, xp,
                    W1_l, W1_r, b1.reshape(1, D))
  (p2,) = _edge_pass(h, srcs, dsts)
  z = _combine_lin(p2.reshape(NC, NPAD, D), degt, h,
                   W2_l, W2_r, b2.reshape(1, D))
  return z[:N]


# speedup vs baseline: 15.0530x; 1.0247x over previous
import jax
import jax.numpy as jnp
from jax import lax
from jax.experimental import pallas as pl
from jax.experimental.pallas import tpu as pltpu
from jax.experimental.pallas import tpu_sc as plsc

N = 10000
D = 128
E = 320000
NC = 2
NS = 16
NW = NC * NS
C = 80
NCHUNK = 128
EPT = C * NCHUNK
EPAD = NW * EPT
NPAD = 10240
RPT = NPAD // NS
NB = 4
ND = 8


def _make_edge_pass(with_deg):
  mesh = plsc.VectorSubcoreMesh(core_axis_name="c", subcore_axis_name="s")
  out_type = [jax.ShapeDtypeStruct((NC * NPAD, D), jnp.float32)]
  scratch = [
      pltpu.VMEM((NB, C), jnp.int32),
      pltpu.VMEM((ND, C), jnp.int32),
      pltpu.VMEM((NB, C, D), jnp.float32),
      pltpu.VMEM_SHARED((NPAD, D), jnp.float32),
  ]
  scratch += [pltpu.SemaphoreType.DMA] * NB
  scratch += [pltpu.SemaphoreType.DMA] * NB
  scratch += [pltpu.SemaphoreType.DMA] * NB
  scratch += [pltpu.SemaphoreType.DMA] * ND
  if with_deg:
    out_type.append(jax.ShapeDtypeStruct((NC * NPAD,), jnp.float32))
    scratch += [
        pltpu.VMEM((C,), jnp.float32),
        pltpu.VMEM((RPT,), jnp.float32),
        pltpu.VMEM_SHARED((NPAD,), jnp.float32),
    ]

  def body(table, srcs, dsts, *refs):
    out_p = refs[0]
    refs = refs[1:]
    if with_deg:
      out_deg = refs[0]
      refs = refs[1:]
    si, di, bufs, acc = refs[0:4]
    gsem = refs[4:4 + NB]
    ssem = refs[4 + NB:4 + 2 * NB]
    isem_s = refs[4 + 2 * NB:4 + 3 * NB]
    isem_d = refs[4 + 3 * NB:4 + 3 * NB + ND]
    if with_deg:
      ones_v, zeros_v, deg_acc = refs[4 + 3 * NB + ND:]
    cid = lax.axis_index("c")
    sid = lax.axis_index("s")
    wid = sid * NC + cid
    base = wid * NCHUNK
    zvec = jnp.zeros((16,), jnp.float32)

    def zero_buf0(t, carry):
      bufs[0, t // 8, pl.ds((t % 8) * 16, 16)] = zvec
      return carry
    lax.fori_loop(0, C * 8, zero_buf0, 0)
    for b in range(RPT // C):
      pltpu.sync_copy(bufs.at[0], acc.at[pl.ds(sid * RPT + b * C, C)])

    if with_deg:
      def init_ones(t, carry):
        ones_v[pl.ds(t * 16, 16)] = zvec + 1.0
        return carry
      lax.fori_loop(0, C // 16, init_ones, 0)

      def zero_zeros(t, carry):
        zeros_v[pl.ds(t * 16, 16)] = zvec
        return carry
      lax.fori_loop(0, RPT // 16, zero_zeros, 0)
      pltpu.sync_copy(zeros_v, deg_acc.at[pl.ds(sid * RPT, RPT)])

    def req_si(c, slot):
      pltpu.async_copy(srcs.at[base + c], si.at[slot], isem_s[slot])

    def wait_si(slot):
      pltpu.make_async_copy(srcs.at[base], si.at[slot], isem_s[slot]).wait()

    def req_di(c, slot):
      pltpu.async_copy(dsts.at[base + c], di.at[slot], isem_d[slot])

    def wait_di(slot):
      pltpu.make_async_copy(dsts.at[base], di.at[slot], isem_d[slot]).wait()

    def start_gather(c, slot):
      pltpu.async_copy(table.at[si.at[slot]], bufs.at[slot], gsem[slot])

    def wait_gather(slot):
      pltpu.make_async_copy(table.at[si.at[slot]], bufs.at[slot],
                            gsem[slot]).wait()

    def start_scatter(sb, dslot):
      pltpu.async_copy(bufs.at[sb], acc.at[di.at[dslot]], ssem[sb], add=True)
      if with_deg:
        pltpu.async_copy(ones_v, deg_acc.at[di.at[dslot]], ssem[sb], add=True)

    def wait_scatter(sb, dslot):
      pltpu.make_async_copy(bufs.at[sb], acc.at[di.at[dslot]],
                            ssem[sb]).wait()
      if with_deg:
        pltpu.make_async_copy(ones_v, deg_acc.at[di.at[dslot]],
                              ssem[sb]).wait()

    def full_step(c, k, first, tail_k=None):
      sb = k & 3
      wait_di(k & 7)
      wait_gather(sb)
      start_scatter(sb, k & 7)
      if tail_k is None or k <= 4:
        wait_si((k + 3) & 3)
        if not (first and k == 0):
          wait_scatter((k + 3) & 3, (k - 1) & 7)
        start_gather(c + 3, (k + 3) & 3)
      if tail_k is None or k <= 3:
        req_si(c + 4, k & 3)
      if (tail_k is None and not (first and k == 0)) or (
          tail_k is not None and k == 0):
        req_di(c + 7, (k - 1) & 7)

    for c in range(NB):
      req_si(c, c)
    for c in range(ND):
      req_di(c, c)
    plsc.subcore_barrier()
    for c in range(3):
      wait_si(c)
      start_gather(c, c)

    for k in range(8):
      full_step(k, k, first=True)

    def octet(o, carry):
      c0 = 8 * o
      for k in range(8):
        full_step(c0 + k, k, first=False)
      return carry
    lax.fori_loop(1, NCHUNK // 8 - 1, octet, 0)

    c0 = NCHUNK - 8
    for k in range(8):
      full_step(c0 + k, k, first=False, tail_k=k)

    for k in range(4, 8):
      wait_scatter(k & 3, k)

    plsc.subcore_barrier()
    row0 = cid * NPAD + sid * RPT
    pltpu.sync_copy(acc.at[pl.ds(sid * RPT, RPT)],
                    out_p.at[pl.ds(row0, RPT)])
    if with_deg:
      pltpu.sync_copy(deg_acc.at[pl.ds(sid * RPT, RPT)],
                      out_deg.at[pl.ds(row0, RPT)])

  return pl.kernel(body, out_type=tuple(out_type), mesh=mesh,
                   scratch_types=tuple(scratch))


_edge_pass_deg = _make_edge_pass(True)
_edge_pass = _make_edge_pass(False)

_R = 2000


def _make_combine(apply_relu):
  def body(p_ref, deg_ref, x_ref, wl_ref, wr_ref, b_ref, o_ref):
    deg = deg_ref[:, 0:1] + deg_ref[:, 1:2]
    invd = 1.0 / jnp.maximum(deg, 1.0)
    mean = (p_ref[0] + p_ref[1]) * invd
    acc = jnp.dot(mean, wl_ref[...], preferred_element_type=jnp.float32)
    acc = acc + jnp.dot(x_ref[...], wr_ref[...],
                        preferred_element_type=jnp.float32)
    acc = acc + b_ref[...]
    if apply_relu:
      acc = jnp.maximum(acc, 0.0)
    o_ref[...] = acc

  return pl.pallas_call(
      body,
      grid=(N // _R,),
      in_specs=[
          pl.BlockSpec((2, _R, D), lambda i: (0, i, 0)),
          pl.BlockSpec((_R, 2), lambda i: (i, 0)),
          pl.BlockSpec((_R, D), lambda i: (i, 0)),
          pl.BlockSpec((D, D), lambda i: (0, 0)),
          pl.BlockSpec((D, D), lambda i: (0, 0)),
          pl.BlockSpec((1, D), lambda i: (0, 0)),
      ],
      out_specs=pl.BlockSpec((_R, D), lambda i: (i, 0)),
      out_shape=jax.ShapeDtypeStruct((N, D), jnp.float32),
  )


_combine_relu = _make_combine(True)
_combine_lin = _make_combine(False)


def kernel(x, edge_index, W1_l, b1, W1_r, W2_l, b2, W2_r):
  src = edge_index[0]
  dst = edge_index[1]
  pad = EPAD - E
  ar = jnp.arange(pad, dtype=jnp.int32)
  psrc = ar % N
  pdst = N + (ar % (NPAD - N))
  srcs = jnp.concatenate([src, psrc]).reshape(NW * NCHUNK, C)
  dsts = jnp.concatenate([dst, pdst]).reshape(NW * NCHUNK, C)

  p1, deg = _edge_pass_deg(x, srcs, dsts)
  degt = deg.reshape(NC, NPAD).T
  h = _combine_relu(p1.reshape(NC, NPAD, D), degt, x,
                    W1_l, W1_r, b1.reshape(1, D))
  (p2,) = _edge_pass(h, srcs, dsts)
  z = _combine_lin(p2.reshape(NC, NPAD, D), degt, h,
                   W2_l, W2_r, b2.reshape(1, D))
  return z


# speedup vs baseline: 15.0921x; 1.0026x over previous
import jax
import jax.numpy as jnp
from jax import lax
from jax.experimental import pallas as pl
from jax.experimental.pallas import tpu as pltpu
from jax.experimental.pallas import tpu_sc as plsc

N = 10000
D = 128
E = 320000
NC = 2
NS = 16
NW = NC * NS
C = 80
NCHUNK = 128
EPT = C * NCHUNK
EPAD = NW * EPT
NPAD = 10240
RPT = NPAD // NS
NB = 4
ND = 8


def _make_edge_pass(with_deg):
  mesh = plsc.VectorSubcoreMesh(core_axis_name="c", subcore_axis_name="s")
  out_type = [jax.ShapeDtypeStruct((NC * NPAD, D), jnp.float32)]
  scratch = [
      pltpu.VMEM((NB, C), jnp.int32),
      pltpu.VMEM((ND, C), jnp.int32),
      pltpu.VMEM((NB, C, D), jnp.float32),
      pltpu.VMEM_SHARED((NPAD, D), jnp.float32),
  ]
  scratch += [pltpu.SemaphoreType.DMA] * NB
  scratch += [pltpu.SemaphoreType.DMA] * NB
  scratch += [pltpu.SemaphoreType.DMA] * NB
  scratch += [pltpu.SemaphoreType.DMA] * ND
  if with_deg:
    out_type.append(jax.ShapeDtypeStruct((NC * NPAD,), jnp.float32))
    scratch += [
        pltpu.VMEM((C,), jnp.float32),
        pltpu.VMEM((RPT,), jnp.float32),
        pltpu.VMEM_SHARED((NPAD,), jnp.float32),
    ]

  def body(table, srcs, dsts, *refs):
    out_p = refs[0]
    refs = refs[1:]
    if with_deg:
      out_deg = refs[0]
      refs = refs[1:]
    si, di, bufs, acc = refs[0:4]
    gsem = refs[4:4 + NB]
    ssem = refs[4 + NB:4 + 2 * NB]
    isem_s = refs[4 + 2 * NB:4 + 3 * NB]
    isem_d = refs[4 + 3 * NB:4 + 3 * NB + ND]
    if with_deg:
      ones_v, zeros_v, deg_acc = refs[4 + 3 * NB + ND:]
    cid = lax.axis_index("c")
    sid = lax.axis_index("s")
    wid = sid * NC + cid
    base = wid * NCHUNK
    zvec = jnp.zeros((16,), jnp.float32)

    def zero_buf0(t, carry):
      bufs[0, t // 8, pl.ds((t % 8) * 16, 16)] = zvec
      return carry
    lax.fori_loop(0, C * 8, zero_buf0, 0)
    for b in range(RPT // C):
      pltpu.sync_copy(bufs.at[0], acc.at[pl.ds(sid * RPT + b * C, C)])

    if with_deg:
      def init_ones(t, carry):
        ones_v[pl.ds(t * 16, 16)] = zvec + 1.0
        return carry
      lax.fori_loop(0, C // 16, init_ones, 0)

      def zero_zeros(t, carry):
        zeros_v[pl.ds(t * 16, 16)] = zvec
        return carry
      lax.fori_loop(0, RPT // 16, zero_zeros, 0)
      pltpu.sync_copy(zeros_v, deg_acc.at[pl.ds(sid * RPT, RPT)])

    def req_si(c, slot):
      pltpu.async_copy(srcs.at[base + c], si.at[slot], isem_s[slot])

    def wait_si(slot):
      pltpu.make_async_copy(srcs.at[base], si.at[slot], isem_s[slot]).wait()

    def req_di(c, slot):
      pltpu.async_copy(dsts.at[base + c], di.at[slot], isem_d[slot])

    def wait_di(slot):
      pltpu.make_async_copy(dsts.at[base], di.at[slot], isem_d[slot]).wait()

    def start_gather(c, slot):
      pltpu.async_copy(table.at[si.at[slot]], bufs.at[slot], gsem[slot])

    def wait_gather(slot):
      pltpu.make_async_copy(table.at[si.at[slot]], bufs.at[slot],
                            gsem[slot]).wait()

    def start_scatter(sb, dslot):
      pltpu.async_copy(bufs.at[sb], acc.at[di.at[dslot]], ssem[sb], add=True)
      if with_deg:
        pltpu.async_copy(ones_v, deg_acc.at[di.at[dslot]], ssem[sb], add=True)

    def wait_scatter(sb, dslot):
      pltpu.make_async_copy(bufs.at[sb], acc.at[di.at[dslot]],
                            ssem[sb]).wait()
      if with_deg:
        pltpu.make_async_copy(ones_v, deg_acc.at[di.at[dslot]],
                              ssem[sb]).wait()

    def full_step(c, k, first, tail_k=None):
      sb = k & 3
      wait_di(k & 7)
      wait_gather(sb)
      start_scatter(sb, k & 7)
      if tail_k is None or k <= 4:
        wait_si((k + 3) & 3)
        if not (first and k == 0):
          wait_scatter((k + 3) & 3, (k - 1) & 7)
        start_gather(c + 3, (k + 3) & 3)
      if tail_k is None or k <= 3:
        req_si(c + 4, k & 3)
      if (tail_k is None and not (first and k == 0)) or (
          tail_k is not None and k == 0):
        req_di(c + 7, (k - 1) & 7)

    for c in range(NB):
      req_si(c, c)
    for c in range(ND):
      req_di(c, c)
    plsc.subcore_barrier()
    for c in range(3):
      wait_si(c)
      start_gather(c, c)

    for k in range(8):
      full_step(k, k, first=True)

    def octet(o, carry):
      c0 = 8 * o
      for k in range(8):
        full_step(c0 + k, k, first=False)
      return carry
    lax.fori_loop(1, NCHUNK // 8 - 1, octet, 0)

    c0 = NCHUNK - 8
    for k in range(8):
      full_step(c0 + k, k, first=False, tail_k=k)

    for k in range(4, 8):
      wait_scatter(k & 3, k)

    plsc.subcore_barrier()
    row0 = cid * NPAD + sid * RPT
    pltpu.sync_copy(acc.at[pl.ds(sid * RPT, RPT)],
                    out_p.at[pl.ds(row0, RPT)])
    if with_deg:
      pltpu.sync_copy(deg_acc.at[pl.ds(sid * RPT, RPT)],
                      out_deg.at[pl.ds(row0, RPT)])

  return pl.kernel(body, out_type=tuple(out_type), mesh=mesh,
                   scratch_types=tuple(scratch))


_edge_pass_deg = _make_edge_pass(True)
_edge_pass = _make_edge_pass(False)

_R = 2000


def _make_rpart():
  def body(x_ref, wr_ref, b_ref, o_ref):
    o_ref[...] = jnp.dot(x_ref[...], wr_ref[...],
                         preferred_element_type=jnp.float32) + b_ref[...]

  return pl.pallas_call(
      body,
      grid=(N // _R,),
      in_specs=[
          pl.BlockSpec((_R, D), lambda i: (i, 0)),
          pl.BlockSpec((D, D), lambda i: (0, 0)),
          pl.BlockSpec((1, D), lambda i: (0, 0)),
      ],
      out_specs=pl.BlockSpec((_R, D), lambda i: (i, 0)),
      out_shape=jax.ShapeDtypeStruct((N, D), jnp.float32),
  )


def _make_combine(apply_relu):
  def body(p_ref, deg_ref, r_ref, wl_ref, o_ref):
    deg = deg_ref[:, 0:1] + deg_ref[:, 1:2]
    invd = 1.0 / jnp.maximum(deg, 1.0)
    mean = (p_ref[0] + p_ref[1]) * invd
    acc = jnp.dot(mean, wl_ref[...], preferred_element_type=jnp.float32)
    acc = acc + r_ref[...]
    if apply_relu:
      acc = jnp.maximum(acc, 0.0)
    o_ref[...] = acc

  return pl.pallas_call(
      body,
      grid=(N // _R,),
      in_specs=[
          pl.BlockSpec((2, _R, D), lambda i: (0, i, 0)),
          pl.BlockSpec((_R, 2), lambda i: (i, 0)),
          pl.BlockSpec((_R, D), lambda i: (i, 0)),
          pl.BlockSpec((D, D), lambda i: (0, 0)),
      ],
      out_specs=pl.BlockSpec((_R, D), lambda i: (i, 0)),
      out_shape=jax.ShapeDtypeStruct((N, D), jnp.float32),
  )


_rpart = _make_rpart()
_combine_relu = _make_combine(True)
_combine_lin = _make_combine(False)


def kernel(x, edge_index, W1_l, b1, W1_r, W2_l, b2, W2_r):
  src = edge_index[0]
  dst = edge_index[1]
  pad = EPAD - E
  ar = jnp.arange(pad, dtype=jnp.int32)
  psrc = ar % N
  pdst = N + (ar % (NPAD - N))
  srcs = jnp.concatenate([src, psrc]).reshape(NW * NCHUNK, C)
  dsts = jnp.concatenate([dst, pdst]).reshape(NW * NCHUNK, C)

  r1 = _rpart(x, W1_r, b1.reshape(1, D))
  p1, deg = _edge_pass_deg(x, srcs, dsts)
  degt = deg.reshape(NC, NPAD).T
  h = _combine_relu(p1.reshape(NC, NPAD, D), degt, r1, W1_l)
  r2 = _rpart(h, W2_r, b2.reshape(1, D))
  (p2,) = _edge_pass(h, srcs, dsts)
  z = _combine_lin(p2.reshape(NC, NPAD, D), degt, r2, W2_l)
  return z
